# baseline (speedup 1.0000x reference)
import jax
import jax.numpy as jnp
from jax import lax
from jax.experimental import pallas as pl
from jax.experimental.pallas import tpu as pltpu
from jax.experimental.pallas import tpu_sc as plsc

_E, _B, _S, _D = 8, 2, 2048, 1024
_K = 2
_PAD_E = 16
_C_BLK = 2048
_NC = _S // _C_BLK
_NEG = -1e30

_E_SC = 1
_E_TC = _E - _E_SC
_NW = 32
_PAIRS = _E_SC * _B
_WPP = _NW // _PAIRS
_RPW = _S // _WPP
_CH = 32
_NCH = _RPW // _CH
_RG = 8


def _scgate_body(hs_hbm, psum_out, buf0, buf1, acc, sem0, sem1):
    cid = lax.axis_index("c")
    sid = lax.axis_index("s")
    wid = sid * 2 + cid
    pair = wid // _WPP
    sub = wid % _WPP
    e = _E_TC + pair // _B
    b = pair % _B
    row0 = (e * _B + b) * _S + sub * _RPW

    zeros = jnp.zeros((16,), jnp.float32)
    for j in range(_D // 16):
        acc[pl.ds(j * 16, 16)] = zeros

    bufs = (buf0, buf1)
    sems = (sem0, sem1)
    copies = {0: pltpu.async_copy(hs_hbm.at[pl.ds(row0, _CH)], buf0, sem0)}
    for c in range(_NCH):
        if c + 1 < _NCH:
            copies[c + 1] = pltpu.async_copy(
                hs_hbm.at[pl.ds(row0 + (c + 1) * _CH, _CH)],
                bufs[(c + 1) % 2], sems[(c + 1) % 2])
        copies[c].wait()
        buf = bufs[c % 2]

        def gbody(gr, carry):
            r0 = gr * _RG
            for j in range(_D // 16):
                sl = pl.ds(j * 16, 16)
                x = [buf[r0 + k, sl] for k in range(_RG)]
                while len(x) > 1:
                    x = [x[i] + x[i + 1] for i in range(0, len(x), 2)]
                plsc.addupdate(acc.at[sl], x[0])
            return carry

        lax.fori_loop(0, _CH // _RG, gbody, 0)

    pltpu.sync_copy(acc, psum_out.at[wid])


_scgate_call = None


def _get_scgate():
    global _scgate_call
    if _scgate_call is None:
        _scgate_call = pl.kernel(
            _scgate_body,
            out_type=jax.ShapeDtypeStruct((_NW, _D), jnp.float32),
            mesh=plsc.VectorSubcoreMesh(core_axis_name="c",
                                        subcore_axis_name="s"),
            scratch_types=[pltpu.VMEM((_CH, _D), jnp.float32),
                           pltpu.VMEM((_CH, _D), jnp.float32),
                           pltpu.VMEM((_D,), jnp.float32),
                           pltpu.SemaphoreType.DMA,
                           pltpu.SemaphoreType.DMA],
        )
    return _scgate_call


def _gate_tc_body(x_ref, w_ref, bias_ref, gate_ref):
    e = pl.program_id(0)
    x = x_ref[0].reshape(_B * _S, _D)
    y = lax.dot_general(x, w_ref[...], (((1,), (0,)), ((), ())),
                        preferred_element_type=jnp.float32)
    for b in range(_B):
        gate_ref[e, b] = (jnp.sum(y[b * _S:(b + 1) * _S]) * (1.0 / _S)
                          + bias_ref[0])


_gate_tc_call = pl.pallas_call(
    _gate_tc_body,
    grid=(_E_TC,),
    in_specs=[
        pl.BlockSpec((1, _B, _S, _D), lambda e: (e, 0, 0, 0)),
        pl.BlockSpec((_D, 1), lambda e: (0, 0)),
        pl.BlockSpec(memory_space=pltpu.SMEM),
    ],
    out_specs=pl.BlockSpec(memory_space=pltpu.SMEM),
    out_shape=jax.ShapeDtypeStruct((_E_TC, _PAD_E), jnp.float32),
)


def _route2_body(gate6_ref, psum_ref, w_ref, bias_ref, idx_ref, score_ref):
    g_sc = {}
    for ei in range(_E_SC):
        for b in range(_B):
            lo = (ei * _B + b) * _WPP
            rows = psum_ref[lo:lo + _WPP, :]
            v = jnp.sum(rows, axis=0, keepdims=True)
            gv = jnp.sum(lax.dot_general(
                v, w_ref[...], (((1,), (0,)), ((), ())),
                preferred_element_type=jnp.float32))
            g_sc[(ei, b)] = gv * (1.0 / _S) + bias_ref[0]

    for b in range(_B):
        g = [gate6_ref[e, b] for e in range(_E_TC)]
        g += [g_sc[(ei, b)] for ei in range(_E_SC)]
        m1 = g[0]
        i1 = jnp.int32(0)
        for ee in range(1, _E):
            better = g[ee] > m1
            i1 = jnp.where(better, jnp.int32(ee), i1)
            m1 = jnp.where(better, g[ee], m1)
        m2 = jnp.float32(_NEG)
        i2 = jnp.int32(0)
        for ee in range(_E):
            better = (g[ee] > m2) & (i1 != ee)
            i2 = jnp.where(better, jnp.int32(ee), i2)
            m2 = jnp.where(better, g[ee], m2)
        ev = jnp.exp(m2 - m1)
        p1 = 1.0 / (1.0 + ev)
        idx_ref[0, b] = i1
        idx_ref[1, b] = i2
        score_ref[0, b] = p1 * (1.0 / _K)
        score_ref[1, b] = (1.0 - p1) * (1.0 / _K)


_route2_call = pl.pallas_call(
    _route2_body,
    in_specs=[
        pl.BlockSpec(memory_space=pltpu.SMEM),
        pl.BlockSpec((_NW, _D), lambda: (0, 0)),
        pl.BlockSpec((_D, 1), lambda: (0, 0)),
        pl.BlockSpec(memory_space=pltpu.SMEM),
    ],
    out_specs=[
        pl.BlockSpec(memory_space=pltpu.SMEM),
        pl.BlockSpec(memory_space=pltpu.SMEM),
    ],
    out_shape=(jax.ShapeDtypeStruct((_K, _PAD_E), jnp.int32),
               jax.ShapeDtypeStruct((_K, _PAD_E), jnp.float32)),
)


def _combine_body(idx_ref, score_ref, x0_ref, x1_ref, o_ref):
    b = pl.program_id(0)
    w0 = score_ref[0, b]
    w1 = score_ref[1, b]
    o_ref[0] = x0_ref[0, 0] * w0 + x1_ref[0, 0] * w1


_combine_call = pl.pallas_call(
    _combine_body,
    grid_spec=pltpu.PrefetchScalarGridSpec(
        num_scalar_prefetch=2,
        grid=(_B, _NC),
        in_specs=[
            pl.BlockSpec((1, 1, _C_BLK, _D),
                         lambda b, s, idx, scr: (idx[0, b], b, s, 0)),
            pl.BlockSpec((1, 1, _C_BLK, _D),
                         lambda b, s, idx, scr: (idx[1, b], b, s, 0)),
        ],
        out_specs=pl.BlockSpec((1, _C_BLK, _D),
                               lambda b, s, idx, scr: (b, s, 0)),
    ),
    out_shape=jax.ShapeDtypeStruct((_B, _S, _D), jnp.float32),
)


def kernel(hidden_states, W, b):
    hs2d = hidden_states.reshape(_E * _B * _S, _D)
    psum = _get_scgate()(hs2d)
    gate6 = _gate_tc_call(hidden_states, W, b)
    idx2, score2 = _route2_call(gate6, psum, W, b)
    return _combine_call(idx2, score2, hidden_states, hidden_states)


# speedup vs baseline: 1.0031x; 1.0031x over previous
import jax
import jax.numpy as jnp
from jax import lax
from jax.experimental import pallas as pl
from jax.experimental.pallas import tpu as pltpu
from jax.experimental.pallas import tpu_sc as plsc

_E, _B, _S, _D = 8, 2, 2048, 1024
_K = 2
_PAD_E = 16
_C_BLK = 1024
_NC = _S // _C_BLK
_NEG = -1e30

_E_SC = 1
_E_TC = _E - _E_SC
_NW = 32
_PAIRS = _E_SC * _B
_WPP = _NW // _PAIRS
_RPW = _S // _WPP
_CH = 32
_NCH = _RPW // _CH
_RG = 8


def _scgate_body(hs_hbm, psum_out, buf0, buf1, acc, sem0, sem1):
    cid = lax.axis_index("c")
    sid = lax.axis_index("s")
    wid = sid * 2 + cid
    pair = wid // _WPP
    sub = wid % _WPP
    e = _E_TC + pair // _B
    b = pair % _B
    row0 = (e * _B + b) * _S + sub * _RPW

    zeros = jnp.zeros((16,), jnp.float32)
    for j in range(_D // 16):
        acc[pl.ds(j * 16, 16)] = zeros

    bufs = (buf0, buf1)
    sems = (sem0, sem1)
    copies = {0: pltpu.async_copy(hs_hbm.at[pl.ds(row0, _CH)], buf0, sem0)}
    for c in range(_NCH):
        if c + 1 < _NCH:
            copies[c + 1] = pltpu.async_copy(
                hs_hbm.at[pl.ds(row0 + (c + 1) * _CH, _CH)],
                bufs[(c + 1) % 2], sems[(c + 1) % 2])
        copies[c].wait()
        buf = bufs[c % 2]

        def gbody(gr, carry):
            r0 = gr * _RG
            for j in range(_D // 16):
                sl = pl.ds(j * 16, 16)
                x = [buf[r0 + k, sl] for k in range(_RG)]
                while len(x) > 1:
                    x = [x[i] + x[i + 1] for i in range(0, len(x), 2)]
                plsc.addupdate(acc.at[sl], x[0])
            return carry

        lax.fori_loop(0, _CH // _RG, gbody, 0)

    pltpu.sync_copy(acc, psum_out.at[wid])


_scgate_call = None


def _get_scgate():
    global _scgate_call
    if _scgate_call is None:
        _scgate_call = pl.kernel(
            _scgate_body,
            out_type=jax.ShapeDtypeStruct((_NW, _D), jnp.float32),
            mesh=plsc.VectorSubcoreMesh(core_axis_name="c",
                                        subcore_axis_name="s"),
            scratch_types=[pltpu.VMEM((_CH, _D), jnp.float32),
                           pltpu.VMEM((_CH, _D), jnp.float32),
                           pltpu.VMEM((_D,), jnp.float32),
                           pltpu.SemaphoreType.DMA,
                           pltpu.SemaphoreType.DMA],
        )
    return _scgate_call


def _gate_tc_body(x_ref, w_ref, bias_ref, gate_ref):
    e = pl.program_id(0)
    x = x_ref[0].reshape(_B * _S, _D)
    y = lax.dot_general(x, w_ref[...], (((1,), (0,)), ((), ())),
                        preferred_element_type=jnp.float32)
    for b in range(_B):
        gate_ref[e, b] = (jnp.sum(y[b * _S:(b + 1) * _S]) * (1.0 / _S)
                          + bias_ref[0])


_gate_tc_call = pl.pallas_call(
    _gate_tc_body,
    grid=(_E_TC,),
    in_specs=[
        pl.BlockSpec((1, _B, _S, _D), lambda e: (e, 0, 0, 0)),
        pl.BlockSpec((_D, 1), lambda e: (0, 0)),
        pl.BlockSpec(memory_space=pltpu.SMEM),
    ],
    out_specs=pl.BlockSpec(memory_space=pltpu.SMEM),
    out_shape=jax.ShapeDtypeStruct((_E_TC, _PAD_E), jnp.float32),
)


def _route2_body(gate6_ref, psum_ref, w_ref, bias_ref, idx_ref, score_ref):
    g_sc = {}
    for ei in range(_E_SC):
        for b in range(_B):
            lo = (ei * _B + b) * _WPP
            rows = psum_ref[lo:lo + _WPP, :]
            v = jnp.sum(rows, axis=0, keepdims=True)
            gv = jnp.sum(lax.dot_general(
                v, w_ref[...], (((1,), (0,)), ((), ())),
                preferred_element_type=jnp.float32))
            g_sc[(ei, b)] = gv * (1.0 / _S) + bias_ref[0]

    for b in range(_B):
        g = [gate6_ref[e, b] for e in range(_E_TC)]
        g += [g_sc[(ei, b)] for ei in range(_E_SC)]
        m1 = g[0]
        i1 = jnp.int32(0)
        for ee in range(1, _E):
            better = g[ee] > m1
            i1 = jnp.where(better, jnp.int32(ee), i1)
            m1 = jnp.where(better, g[ee], m1)
        m2 = jnp.float32(_NEG)
        i2 = jnp.int32(0)
        for ee in range(_E):
            better = (g[ee] > m2) & (i1 != ee)
            i2 = jnp.where(better, jnp.int32(ee), i2)
            m2 = jnp.where(better, g[ee], m2)
        ev = jnp.exp(m2 - m1)
        p1 = 1.0 / (1.0 + ev)
        idx_ref[0, b] = i1
        idx_ref[1, b] = i2
        score_ref[0, b] = p1 * (1.0 / _K)
        score_ref[1, b] = (1.0 - p1) * (1.0 / _K)


_route2_call = pl.pallas_call(
    _route2_body,
    in_specs=[
        pl.BlockSpec(memory_space=pltpu.SMEM),
        pl.BlockSpec((_NW, _D), lambda: (0, 0)),
        pl.BlockSpec((_D, 1), lambda: (0, 0)),
        pl.BlockSpec(memory_space=pltpu.SMEM),
    ],
    out_specs=[
        pl.BlockSpec(memory_space=pltpu.SMEM),
        pl.BlockSpec(memory_space=pltpu.SMEM),
    ],
    out_shape=(jax.ShapeDtypeStruct((_K, _PAD_E), jnp.int32),
               jax.ShapeDtypeStruct((_K, _PAD_E), jnp.float32)),
)


def _combine_body(idx_ref, score_ref, x0_ref, x1_ref, o_ref):
    b = pl.program_id(0)
    w0 = score_ref[0, b]
    w1 = score_ref[1, b]
    o_ref[0] = x0_ref[0, 0] * w0 + x1_ref[0, 0] * w1


_combine_call = pl.pallas_call(
    _combine_body,
    grid_spec=pltpu.PrefetchScalarGridSpec(
        num_scalar_prefetch=2,
        grid=(_B, _NC),
        in_specs=[
            pl.BlockSpec((1, 1, _C_BLK, _D),
                         lambda b, s, idx, scr: (idx[0, b], b, s, 0)),
            pl.BlockSpec((1, 1, _C_BLK, _D),
                         lambda b, s, idx, scr: (idx[1, b], b, s, 0)),
        ],
        out_specs=pl.BlockSpec((1, _C_BLK, _D),
                               lambda b, s, idx, scr: (b, s, 0)),
    ),
    out_shape=jax.ShapeDtypeStruct((_B, _S, _D), jnp.float32),
)


def kernel(hidden_states, W, b):
    hs2d = hidden_states.reshape(_E * _B * _S, _D)
    psum = _get_scgate()(hs2d)
    gate6 = _gate_tc_call(hidden_states, W, b)
    idx2, score2 = _route2_call(gate6, psum, W, b)
    return _combine_call(idx2, score2, hidden_states, hidden_states)
